# baseline (speedup 1.0000x reference)
import functools

import jax
import jax.numpy as jnp
from jax import lax
from jax.experimental import pallas as pl
from jax.experimental.pallas import tpu as pltpu
from jax.experimental.pallas import tpu_sc as plsc

_EPS = 1e-12


def _argmax_body(x_ref, cb_ref, idx_ref, cbn_scr, *, kc, tm):
    K = cb_ref.shape[0]

    @pl.when(pl.program_id(0) == 0)
    def _init():
        c = cb_ref[...]
        n = jnp.sqrt(jnp.sum(c * c, axis=1, keepdims=True))
        cbn_scr[...] = c / jnp.maximum(n, _EPS)

    xb = x_ref[...]
    xn = xb / jnp.maximum(jnp.sqrt(jnp.sum(xb * xb, axis=1, keepdims=True)), _EPS)
    nchunk = K // kc
    colsf = lax.broadcasted_iota(jnp.int32, (1, kc), 1).astype(jnp.float32)
    best_v = jnp.full((tm, 1), -jnp.inf, jnp.float32)
    best_i = jnp.zeros((tm, 1), jnp.float32)
    for c in range(nchunk):
        cb = cbn_scr[pl.ds(c * kc, kc), :]
        s = lax.dot_general(xn, cb, (((1,), (1,)), ((), ())),
                            preferred_element_type=jnp.float32)
        m = jnp.max(s, axis=1, keepdims=True)
        li = jnp.min(jnp.where(s >= m, colsf, float(K)), axis=1, keepdims=True)
        upd = m > best_v
        best_i = jnp.where(upd, li + float(c * kc), best_i)
        best_v = jnp.where(upd, m, best_v)
    idx_ref[...] = best_i.astype(jnp.int32).reshape(1, 1, tm)


def _argmax_sims(x_flat, cb):
    M, D = x_flat.shape
    K = cb.shape[0]
    tm, kc = 512, 2048
    return pl.pallas_call(
        functools.partial(_argmax_body, kc=kc, tm=tm),
        grid=(M // tm,),
        in_specs=[
            pl.BlockSpec((tm, D), lambda i: (i, 0)),
            pl.BlockSpec((K, D), lambda i: (0, 0)),
        ],
        out_specs=pl.BlockSpec((1, 1, tm), lambda i: (i, 0, 0)),
        out_shape=jax.ShapeDtypeStruct((M // tm, 1, tm), jnp.int32),
        scratch_shapes=[pltpu.VMEM((K, D), jnp.float32)],
    )(x_flat, cb)


def _make_sc_gather(K, D, M):
    info = plsc.get_sparse_core_info()
    NC, NS = info.num_cores, info.num_subcores
    NW = NC * NS
    rows_w = M // NW
    chunk = 96
    nchunks = rows_w // chunk
    assert rows_w % chunk == 0 and (chunk * D * 4) % 64 == 0
    mesh = plsc.VectorSubcoreMesh(core_axis_name="c", subcore_axis_name="s")

    @functools.partial(
        pl.kernel, mesh=mesh,
        out_type=jax.ShapeDtypeStruct((M, D), jnp.float32),
        scratch_types=[
            pltpu.VMEM((chunk,), jnp.int32),
            pltpu.VMEM((chunk, D), jnp.float32),
            pltpu.SemaphoreType.DMA,
        ],
    )
    def gather_k(cb_hbm, idx_hbm, out_hbm, idx_v, rows_v, sem):
        wid = lax.axis_index("s") * NC + lax.axis_index("c")
        for c in range(nchunks):
            base = (wid * nchunks + c) * chunk
            pltpu.sync_copy(idx_hbm.at[pl.ds(base, chunk)], idx_v)
            pltpu.async_copy(cb_hbm.at[idx_v], rows_v, sem).wait()
            pltpu.sync_copy(rows_v, out_hbm.at[pl.ds(base, chunk)])

    return gather_k


def kernel(x, codebook):
    Bs, Ts, D = x.shape
    K = codebook.shape[0]
    x_flat = x.reshape(-1, D)
    M = x_flat.shape[0]
    idx = _argmax_sims(x_flat, codebook).reshape(-1)
    z_q = _make_sc_gather(K, D, M)(codebook, idx)
    return z_q.reshape(Bs, Ts, D), idx.reshape(Bs, Ts)


# speedup vs baseline: 1.1680x; 1.1680x over previous
import functools

import jax
import jax.numpy as jnp
from jax import lax
from jax.experimental import pallas as pl
from jax.experimental.pallas import tpu as pltpu
from jax.experimental.pallas import tpu_sc as plsc

_EPS = 1e-12


def _argmax_body(x_ref, cb_ref, idx_ref, cbn_scr, *, kc, tm):
    K = cb_ref.shape[0]

    @pl.when(pl.program_id(0) == 0)
    def _init():
        c = cb_ref[...]
        n = jnp.sqrt(jnp.sum(c * c, axis=1, keepdims=True))
        cbn_scr[...] = c / jnp.maximum(n, _EPS)

    xb = x_ref[...]
    xn = xb / jnp.maximum(jnp.sqrt(jnp.sum(xb * xb, axis=1, keepdims=True)), _EPS)
    nchunk = K // kc
    colsf = lax.broadcasted_iota(jnp.int32, (1, kc), 1).astype(jnp.float32)
    best_v = jnp.full((tm, 1), -jnp.inf, jnp.float32)
    best_i = jnp.zeros((tm, 1), jnp.float32)
    for c in range(nchunk):
        cb = cbn_scr[pl.ds(c * kc, kc), :]
        s = lax.dot_general(xn, cb, (((1,), (1,)), ((), ())),
                            preferred_element_type=jnp.float32)
        m = jnp.max(s, axis=1, keepdims=True)
        li = jnp.min(jnp.where(s >= m, colsf, float(K)), axis=1, keepdims=True)
        upd = m > best_v
        best_i = jnp.where(upd, li + float(c * kc), best_i)
        best_v = jnp.where(upd, m, best_v)
    idx_ref[...] = best_i.astype(jnp.int32)


def _argmax_sims(x_flat, cb):
    M, D = x_flat.shape
    K = cb.shape[0]
    tm, kc = 768, 512
    return pl.pallas_call(
        functools.partial(_argmax_body, kc=kc, tm=tm),
        grid=(M // tm,),
        in_specs=[
            pl.BlockSpec((tm, D), lambda i: (i, 0)),
            pl.BlockSpec((K, D), lambda i: (0, 0)),
        ],
        out_specs=pl.BlockSpec((tm, 1), lambda i: (i, 0)),
        out_shape=jax.ShapeDtypeStruct((M, 1), jnp.int32),
        scratch_shapes=[pltpu.VMEM((K, D), jnp.float32)],
    )(x_flat, cb)


def _make_sc_gather(K, D, M):
    info = plsc.get_sparse_core_info()
    NC, NS = info.num_cores, info.num_subcores
    NW = NC * NS
    rows_w = M // NW
    chunk = 96
    nchunks = rows_w // chunk
    assert rows_w % chunk == 0 and (chunk * D * 4) % 64 == 0
    mesh = plsc.VectorSubcoreMesh(core_axis_name="c", subcore_axis_name="s")

    @functools.partial(
        pl.kernel, mesh=mesh,
        out_type=jax.ShapeDtypeStruct((M, D), jnp.float32),
        scratch_types=[
            pltpu.VMEM((chunk,), jnp.int32),
            pltpu.VMEM((chunk, D), jnp.float32),
            pltpu.SemaphoreType.DMA,
        ],
    )
    def gather_k(cb_hbm, idx_hbm, out_hbm, idx_v, rows_v, sem):
        wid = lax.axis_index("s") * NC + lax.axis_index("c")
        for c in range(nchunks):
            base = (wid * nchunks + c) * chunk
            pltpu.sync_copy(idx_hbm.at[pl.ds(base, chunk)], idx_v)
            pltpu.async_copy(cb_hbm.at[idx_v], rows_v, sem).wait()
            pltpu.sync_copy(rows_v, out_hbm.at[pl.ds(base, chunk)])

    return gather_k


def kernel(x, codebook):
    Bs, Ts, D = x.shape
    K = codebook.shape[0]
    x_flat = x.reshape(-1, D)
    M = x_flat.shape[0]
    idx = _argmax_sims(x_flat, codebook).reshape(-1)
    z_q = _make_sc_gather(K, D, M)(codebook, idx)
    return z_q.reshape(Bs, Ts, D), idx.reshape(Bs, Ts)


# speedup vs baseline: 1.1866x; 1.0160x over previous
import functools

import jax
import jax.numpy as jnp
from jax import lax
from jax.experimental import pallas as pl
from jax.experimental.pallas import tpu as pltpu
from jax.experimental.pallas import tpu_sc as plsc

_EPS = 1e-12


def _argmax_body(x_ref, cb_ref, idx_ref, cbn_scr, *, kc, tm):
    K = cb_ref.shape[0]

    @pl.when(pl.program_id(0) == 0)
    def _init():
        c = cb_ref[...]
        n = jnp.sqrt(jnp.sum(c * c, axis=1, keepdims=True))
        cbn_scr[...] = c / jnp.maximum(n, _EPS)

    xb = x_ref[...]
    xn = xb / jnp.maximum(jnp.sqrt(jnp.sum(xb * xb, axis=1, keepdims=True)), _EPS)
    nchunk = K // kc
    colsf = lax.broadcasted_iota(jnp.int32, (1, kc), 1).astype(jnp.float32)
    best_v = jnp.full((tm, 1), -jnp.inf, jnp.float32)
    best_i = jnp.zeros((tm, 1), jnp.float32)
    for c in range(nchunk):
        cb = cbn_scr[pl.ds(c * kc, kc), :]
        s = lax.dot_general(xn, cb, (((1,), (1,)), ((), ())),
                            preferred_element_type=jnp.float32)
        m = jnp.max(s, axis=1, keepdims=True)
        li = jnp.min(jnp.where(s >= m, colsf, float(K)), axis=1, keepdims=True)
        upd = m > best_v
        best_i = jnp.where(upd, li + float(c * kc), best_i)
        best_v = jnp.where(upd, m, best_v)
    idx_ref[...] = best_i.astype(jnp.int32)


def _argmax_sims(x_flat, cb):
    M, D = x_flat.shape
    K = cb.shape[0]
    tm, kc = 1152, 512
    return pl.pallas_call(
        functools.partial(_argmax_body, kc=kc, tm=tm),
        grid=(M // tm,),
        in_specs=[
            pl.BlockSpec((tm, D), lambda i: (i, 0)),
            pl.BlockSpec((K, D), lambda i: (0, 0)),
        ],
        out_specs=pl.BlockSpec((tm, 1), lambda i: (i, 0)),
        out_shape=jax.ShapeDtypeStruct((M, 1), jnp.int32),
        scratch_shapes=[pltpu.VMEM((K, D), jnp.float32)],
    )(x_flat, cb)


def _make_sc_gather(K, D, M):
    info = plsc.get_sparse_core_info()
    NC, NS = info.num_cores, info.num_subcores
    NW = NC * NS
    rows_w = M // NW
    chunk = 96
    nchunks = rows_w // chunk
    assert rows_w % chunk == 0 and (chunk * D * 4) % 64 == 0
    mesh = plsc.VectorSubcoreMesh(core_axis_name="c", subcore_axis_name="s")

    @functools.partial(
        pl.kernel, mesh=mesh,
        out_type=jax.ShapeDtypeStruct((M, D), jnp.float32),
        scratch_types=[
            pltpu.VMEM((chunk,), jnp.int32),
            pltpu.VMEM((chunk, D), jnp.float32),
            pltpu.SemaphoreType.DMA,
        ],
    )
    def gather_k(cb_hbm, idx_hbm, out_hbm, idx_v, rows_v, sem):
        wid = lax.axis_index("s") * NC + lax.axis_index("c")
        for c in range(nchunks):
            base = (wid * nchunks + c) * chunk
            pltpu.sync_copy(idx_hbm.at[pl.ds(base, chunk)], idx_v)
            pltpu.async_copy(cb_hbm.at[idx_v], rows_v, sem).wait()
            pltpu.sync_copy(rows_v, out_hbm.at[pl.ds(base, chunk)])

    return gather_k


def kernel(x, codebook):
    Bs, Ts, D = x.shape
    K = codebook.shape[0]
    x_flat = x.reshape(-1, D)
    M = x_flat.shape[0]
    idx = _argmax_sims(x_flat, codebook).reshape(-1)
    z_q = _make_sc_gather(K, D, M)(codebook, idx)
    return z_q.reshape(Bs, Ts, D), idx.reshape(Bs, Ts)
